# Initial kernel scaffold; baseline (speedup 1.0000x reference)
import functools

import jax
import jax.numpy as jnp
from jax import lax
from jax.experimental import pallas as pl
from jax.experimental.pallas import tpu as pltpu
from jax.experimental.pallas import tpu_sc as plsc

SEQ_LEN = 200
EMBED_DIM = 32
BATCH = 4096

NUM_CORES = 2
NUM_SUBCORES = 16
NUM_WORKERS = NUM_CORES * NUM_SUBCORES

ROWS = BATCH * SEQ_LEN
ROWS_PER_WORKER = ROWS // NUM_WORKERS
SEQS_PER_CHUNK = 4
CHUNK = SEQS_PER_CHUNK * SEQ_LEN
NUM_CHUNKS = ROWS_PER_WORKER // CHUNK


def _body(inputs_hbm, table_hbm, pos_hbm, out_hbm,
          pos_v, idx_v, rows_v, gsem, psem):
    wid = lax.axis_index("s") * NUM_CORES + lax.axis_index("c")
    base = wid * ROWS_PER_WORKER

    pltpu.sync_copy(pos_hbm, pos_v)
    pltpu.async_copy(inputs_hbm.at[pl.ds(base, ROWS_PER_WORKER)], idx_v,
                     psem).wait()

    def chunk_body(g, carry):
        start = base + g * CHUNK
        pltpu.async_copy(table_hbm.at[idx_v.at[pl.ds(g * CHUNK, CHUNK)]],
                         rows_v, gsem).wait()

        def pos_body(p, c):
            p0 = pos_v[p, pl.ds(0, 16)]
            p1 = pos_v[p, pl.ds(16, 16)]
            for s in range(SEQS_PER_CHUNK):
                r = s * SEQ_LEN + p
                rows_v[r, pl.ds(0, 16)] += p0
                rows_v[r, pl.ds(16, 16)] += p1
            return c

        lax.fori_loop(0, SEQ_LEN, pos_body, 0, unroll=False)

        pltpu.sync_copy(rows_v, out_hbm.at[pl.ds(start, CHUNK)])
        return carry

    lax.fori_loop(0, NUM_CHUNKS, chunk_body, 0, unroll=False)


@jax.jit
def kernel(inputs, table, pos):
    flat_idx = inputs.reshape(ROWS)
    mesh = plsc.VectorSubcoreMesh(core_axis_name="c", subcore_axis_name="s")
    out = pl.kernel(
        _body,
        out_type=jax.ShapeDtypeStruct((ROWS, EMBED_DIM), jnp.float32),
        mesh=mesh,
        scratch_types=[
            pltpu.VMEM((SEQ_LEN, EMBED_DIM), jnp.float32),
            pltpu.VMEM((ROWS_PER_WORKER,), jnp.int32),
            pltpu.VMEM((CHUNK, EMBED_DIM), jnp.float32),
            pltpu.SemaphoreType.DMA,
            pltpu.SemaphoreType.DMA,
        ],
    )(flat_idx, table, pos)
    return out.reshape(BATCH, SEQ_LEN, EMBED_DIM)


# speedup vs baseline: 1.4159x; 1.4159x over previous
import functools

import jax
import jax.numpy as jnp
from jax import lax
from jax.experimental import pallas as pl
from jax.experimental.pallas import tpu as pltpu
from jax.experimental.pallas import tpu_sc as plsc

SEQ_LEN = 200
EMBED_DIM = 32
BATCH = 4096

NUM_CORES = 2
NUM_SUBCORES = 16
NUM_WORKERS = NUM_CORES * NUM_SUBCORES

ROWS = BATCH * SEQ_LEN
ROWS_PER_WORKER = ROWS // NUM_WORKERS
SEQS_PER_CHUNK = 4
CHUNK = SEQS_PER_CHUNK * SEQ_LEN
NUM_CHUNKS = ROWS_PER_WORKER // CHUNK


def _body(inputs_hbm, table_hbm, pos_hbm, out_hbm,
          pos_v, idx_v, rows_v, gsem, psem):
    wid = lax.axis_index("s") * NUM_CORES + lax.axis_index("c")
    base = wid * ROWS_PER_WORKER

    pltpu.sync_copy(pos_hbm, pos_v)
    pltpu.async_copy(inputs_hbm.at[pl.ds(base, ROWS_PER_WORKER)], idx_v,
                     psem).wait()

    def chunk_body(g, carry):
        start = base + g * CHUNK
        pltpu.async_copy(table_hbm.at[idx_v.at[pl.ds(g * CHUNK, CHUNK)]],
                         rows_v, gsem).wait()

        def pos_body(p, c):
            p0 = pos_v[p, pl.ds(0, 16)]
            p1 = pos_v[p, pl.ds(16, 16)]
            for s in range(SEQS_PER_CHUNK):
                r = s * SEQ_LEN + p
                rows_v[r, pl.ds(0, 16)] += p0
                rows_v[r, pl.ds(16, 16)] += p1
            return c

        lax.fori_loop(0, SEQ_LEN, pos_body, 0, unroll=False)

        pltpu.sync_copy(rows_v, out_hbm.at[pl.ds(start, CHUNK)])
        return carry

    lax.fori_loop(0, NUM_CHUNKS, chunk_body, 0, unroll=False)


@jax.jit
def kernel(inputs, table, pos):
    flat_idx = inputs.reshape(ROWS)
    mesh = plsc.VectorSubcoreMesh(core_axis_name="c", subcore_axis_name="s")
    out = pl.kernel(
        _body,
        out_type=jax.ShapeDtypeStruct((ROWS, EMBED_DIM), jnp.float32),
        mesh=mesh,
        compiler_params=pltpu.CompilerParams(use_tc_tiling_on_sc=False),
        scratch_types=[
            pltpu.VMEM((SEQ_LEN, EMBED_DIM), jnp.float32),
            pltpu.VMEM((ROWS_PER_WORKER,), jnp.int32),
            pltpu.VMEM((CHUNK, EMBED_DIM), jnp.float32),
            pltpu.SemaphoreType.DMA,
            pltpu.SemaphoreType.DMA,
        ],
    )(flat_idx, table, pos)
    return out.reshape(BATCH, SEQ_LEN, EMBED_DIM)
